# baseline (speedup 1.0000x reference)
import functools

import jax
import jax.numpy as jnp
from jax import lax
from jax.experimental import pallas as pl
from jax.experimental.pallas import tpu as pltpu
from jax.experimental.pallas import tpu_sc as plsc

SEQ_PAD = 256
ROWS_PER_TILE = 16
NUM_TILES = 16
EMBED_DIM = 64
LANES = 16


def _sc_kernel(table_hbm, idx_hbm, wt_hbm, b_hbm, out_hbm,
               idx_v, rows_v, lmax_v, shared, all_v, wt_v, b_v, out_v, sem):
    cid = lax.axis_index("c")
    sid = lax.axis_index("s")

    @pl.when(cid == 0)
    def _gather_and_local_max():
        base = sid * ROWS_PER_TILE
        pltpu.sync_copy(idx_hbm.at[pl.ds(base, ROWS_PER_TILE)], idx_v)
        pltpu.async_copy(table_hbm.at[idx_v], rows_v, sem).wait()
        for c in range(EMBED_DIM // LANES):
            m = rows_v[0, pl.ds(c * LANES, LANES)]
            for r in range(1, ROWS_PER_TILE):
                m = jnp.maximum(m, rows_v[r, pl.ds(c * LANES, LANES)])
            lmax_v[pl.ds(c * LANES, LANES)] = m
        pltpu.sync_copy(lmax_v, shared.at[sid])

    plsc.subcore_barrier()

    @pl.when(jnp.logical_and(cid == 0, sid == 0))
    def _reduce_and_linear():
        pltpu.sync_copy(shared, all_v)
        pltpu.sync_copy(wt_hbm, wt_v)
        pltpu.sync_copy(b_hbm, b_v)
        acc = b_v[...]
        for c in range(EMBED_DIM // LANES):
            g = all_v[0, pl.ds(c * LANES, LANES)]
            for s in range(1, NUM_TILES):
                g = jnp.maximum(g, all_v[s, pl.ds(c * LANES, LANES)])
            for l in range(LANES):
                acc = acc + g[l] * wt_v[c * LANES + l, :]
        out_v[...] = acc
        pltpu.sync_copy(out_v, out_hbm)


@jax.jit
def _run(xp, table, wt, bp):
    mesh = plsc.VectorSubcoreMesh(core_axis_name="c", subcore_axis_name="s")
    k = functools.partial(
        pl.kernel,
        mesh=mesh,
        out_type=jax.ShapeDtypeStruct((LANES,), jnp.float32),
        compiler_params=pltpu.CompilerParams(use_tc_tiling_on_sc=False),
        scratch_types=[
            pltpu.VMEM((ROWS_PER_TILE,), jnp.int32),
            pltpu.VMEM((ROWS_PER_TILE, EMBED_DIM), jnp.float32),
            pltpu.VMEM((EMBED_DIM,), jnp.float32),
            pltpu.VMEM_SHARED((NUM_TILES, EMBED_DIM), jnp.float32),
            pltpu.VMEM((NUM_TILES, EMBED_DIM), jnp.float32),
            pltpu.VMEM((EMBED_DIM, LANES), jnp.float32),
            pltpu.VMEM((LANES,), jnp.float32),
            pltpu.VMEM((LANES,), jnp.float32),
            pltpu.SemaphoreType.DMA,
        ],
    )(_sc_kernel)
    return k(table, xp, wt, bp)


def kernel(x, table, W, b):
    xf = x.reshape(-1)
    xp = jnp.concatenate(
        [xf, jnp.broadcast_to(xf[0], (SEQ_PAD - xf.shape[0],))]).astype(jnp.int32)
    wt = jnp.zeros((EMBED_DIM, LANES), jnp.float32).at[:, :W.shape[0]].set(W.T)
    bp = jnp.zeros((LANES,), jnp.float32).at[:b.shape[0]].set(b)
    res = _run(xp, table, wt, bp)
    return res[:W.shape[0]].reshape(1, -1)


# speedup vs baseline: 2.5424x; 2.5424x over previous
import functools

import jax
import jax.numpy as jnp
from jax import lax
from jax.experimental import pallas as pl
from jax.experimental.pallas import tpu as pltpu
from jax.experimental.pallas import tpu_sc as plsc

SEQ_PAD = 256
ROWS_PER_TILE = 16
NUM_TILES = 16
EMBED_DIM = 64
LANES = 16
BLK = 8
NCHUNK = EMBED_DIM // LANES


def _sc_kernel(table_hbm, blk_hbm, sub_hbm, wt_hbm, b_hbm, out_hbm,
               bidx_v, sidx_v, blocks_v, lmax_v, tmp_v, wt_v, b_v, out_v,
               sem, *shared):
    cid = lax.axis_index("c")
    sid = lax.axis_index("s")

    @pl.when(cid == 0)
    def _gather_and_local_max():
        base = sid * ROWS_PER_TILE
        pltpu.sync_copy(blk_hbm.at[pl.ds(base, ROWS_PER_TILE)], bidx_v)
        pltpu.sync_copy(sub_hbm.at[pl.ds(base, ROWS_PER_TILE)], sidx_v)
        bv = bidx_v[...]
        handles = [
            pltpu.async_copy(table_hbm.at[bv[r]], blocks_v.at[r], sem)
            for r in range(ROWS_PER_TILE)
        ]
        for h in handles:
            h.wait()
        sv = sidx_v[...]
        lanes = lax.iota(jnp.int32, LANES)
        acc = [None] * NCHUNK
        for r in range(ROWS_PER_TILE):
            i0 = jnp.broadcast_to(jnp.int32(r), (LANES,))
            i1 = jnp.broadcast_to(sv[r], (LANES,))
            for c in range(NCHUNK):
                g = plsc.load_gather(blocks_v, [i0, i1, c * LANES + lanes])
                acc[c] = g if acc[c] is None else jnp.maximum(acc[c], g)
        for c in range(NCHUNK):
            lmax_v[pl.ds(c * LANES, LANES)] = acc[c]
        for k in range(NUM_TILES):
            @pl.when(sid == k)
            def _stage(k=k):
                pltpu.sync_copy(lmax_v, shared[k])

    plsc.subcore_barrier()

    @pl.when(jnp.logical_and(cid == 0, sid == 0))
    def _reduce_and_linear():
        pltpu.sync_copy(wt_hbm, wt_v)
        pltpu.sync_copy(b_hbm, b_v)
        gacc = [None] * NCHUNK
        for k in range(NUM_TILES):
            pltpu.sync_copy(shared[k], tmp_v)
            for c in range(NCHUNK):
                t = tmp_v[pl.ds(c * LANES, LANES)]
                gacc[c] = t if gacc[c] is None else jnp.maximum(gacc[c], t)
        acc = b_v[...]
        for c in range(NCHUNK):
            g = gacc[c]
            for l in range(LANES):
                acc = acc + g[l] * wt_v[c * LANES + l, :]
        out_v[...] = acc
        pltpu.sync_copy(out_v, out_hbm)


@jax.jit
def _run(xb, xs, table3, wt, bp):
    mesh = plsc.VectorSubcoreMesh(core_axis_name="c", subcore_axis_name="s")
    k = functools.partial(
        pl.kernel,
        mesh=mesh,
        out_type=jax.ShapeDtypeStruct((LANES,), jnp.float32),
        compiler_params=pltpu.CompilerParams(needs_layout_passes=False),
        scratch_types=[
            pltpu.VMEM((ROWS_PER_TILE,), jnp.int32),
            pltpu.VMEM((ROWS_PER_TILE,), jnp.int32),
            pltpu.VMEM((ROWS_PER_TILE, BLK, EMBED_DIM), jnp.float32),
            pltpu.VMEM((EMBED_DIM,), jnp.float32),
            pltpu.VMEM((EMBED_DIM,), jnp.float32),
            pltpu.VMEM((EMBED_DIM, LANES), jnp.float32),
            pltpu.VMEM((LANES,), jnp.float32),
            pltpu.VMEM((LANES,), jnp.float32),
            pltpu.SemaphoreType.DMA,
        ] + [pltpu.VMEM_SHARED((EMBED_DIM,), jnp.float32)
             for _ in range(NUM_TILES)],
    )(_sc_kernel)
    return k(table3, xb, xs, wt, bp)


def kernel(x, table, W, b):
    xf = x.reshape(-1)
    xp = jnp.concatenate(
        [xf, jnp.broadcast_to(xf[0], (SEQ_PAD - xf.shape[0],))]).astype(jnp.int32)
    xb = xp // BLK
    xs = xp % BLK
    table3 = table.reshape(table.shape[0] // BLK, BLK, EMBED_DIM)
    wt = jnp.zeros((EMBED_DIM, LANES), jnp.float32).at[:, :W.shape[0]].set(W.T)
    bp = jnp.zeros((LANES,), jnp.float32).at[:b.shape[0]].set(b)
    res = _run(xb, xs, table3, wt, bp)
    return res[:W.shape[0]].reshape(1, -1)


# speedup vs baseline: 16.3060x; 6.4136x over previous
import functools

import jax
import jax.numpy as jnp
from jax import lax
from jax.experimental import pallas as pl
from jax.experimental.pallas import tpu as pltpu
from jax.experimental.pallas import tpu_sc as plsc

SEQ_PAD = 256
IDX_PER_TILE = 16
NUM_TILES = 16
EMBED_DIM = 64
LANES = 16
SLAB = 128
WAVE = 8
NCHUNK = EMBED_DIM // LANES


def _sc_kernel(tt_hbm, blk_hbm, sub_hbm, wt_hbm, b_hbm, out_hbm,
               bidx_v, sidx_v, slabs_v, lmax_v, tmp_v, wt_v, b_v, out_v,
               sem, *shared):
    cid = lax.axis_index("c")
    sid = lax.axis_index("s")

    @pl.when(cid == 0)
    def _gather_and_local_max():
        base = sid * IDX_PER_TILE
        pltpu.sync_copy(blk_hbm.at[pl.ds(base, IDX_PER_TILE)], bidx_v)
        pltpu.sync_copy(sub_hbm.at[pl.ds(base, IDX_PER_TILE)], sidx_v)
        bv = bidx_v[...]
        sv = sidx_v[...]
        lanes = lax.iota(jnp.int32, LANES)
        acc = [None] * NCHUNK
        for w in range(IDX_PER_TILE // WAVE):
            handles = [
                pltpu.async_copy(
                    tt_hbm.at[:, pl.ds(
                        pl.multiple_of(bv[w * WAVE + q], SLAB), SLAB)],
                    slabs_v.at[q], sem)
                for q in range(WAVE)
            ]
            for h in handles:
                h.wait()
            for q in range(WAVE):
                i0 = jnp.broadcast_to(jnp.int32(q), (LANES,))
                i2 = jnp.broadcast_to(sv[w * WAVE + q], (LANES,))
                for c in range(NCHUNK):
                    g = plsc.load_gather(slabs_v, [i0, c * LANES + lanes, i2])
                    acc[c] = g if acc[c] is None else jnp.maximum(acc[c], g)
        for c in range(NCHUNK):
            lmax_v[pl.ds(c * LANES, LANES)] = acc[c]
        for k in range(NUM_TILES):
            @pl.when(sid == k)
            def _stage(k=k):
                pltpu.sync_copy(lmax_v, shared[k])

    plsc.subcore_barrier()

    @pl.when(jnp.logical_and(cid == 0, sid == 0))
    def _reduce_and_linear():
        pltpu.sync_copy(wt_hbm, wt_v)
        pltpu.sync_copy(b_hbm, b_v)
        gacc = [None] * NCHUNK
        for k in range(NUM_TILES):
            pltpu.sync_copy(shared[k], tmp_v)
            for c in range(NCHUNK):
                t = tmp_v[pl.ds(c * LANES, LANES)]
                gacc[c] = t if gacc[c] is None else jnp.maximum(gacc[c], t)
        acc = b_v[...]
        for c in range(NCHUNK):
            g = gacc[c]
            for l in range(LANES):
                acc = acc + g[l] * wt_v[c * LANES + l, :]
        out_v[...] = acc
        pltpu.sync_copy(out_v, out_hbm)


@jax.jit
def _run(xb, xs, tt, wt, bp):
    mesh = plsc.VectorSubcoreMesh(core_axis_name="c", subcore_axis_name="s")
    k = functools.partial(
        pl.kernel,
        mesh=mesh,
        out_type=jax.ShapeDtypeStruct((LANES,), jnp.float32),
        compiler_params=pltpu.CompilerParams(needs_layout_passes=False),
        scratch_types=[
            pltpu.VMEM((IDX_PER_TILE,), jnp.int32),
            pltpu.VMEM((IDX_PER_TILE,), jnp.int32),
            pltpu.VMEM((WAVE, EMBED_DIM, SLAB), jnp.float32),
            pltpu.VMEM((EMBED_DIM,), jnp.float32),
            pltpu.VMEM((EMBED_DIM,), jnp.float32),
            pltpu.VMEM((EMBED_DIM, LANES), jnp.float32),
            pltpu.VMEM((LANES,), jnp.float32),
            pltpu.VMEM((LANES,), jnp.float32),
            pltpu.SemaphoreType.DMA,
        ] + [pltpu.VMEM_SHARED((EMBED_DIM,), jnp.float32)
             for _ in range(NUM_TILES)],
    )(_sc_kernel)
    return k(tt, xb, xs, wt, bp)


def kernel(x, table, W, b):
    xf = x.reshape(-1)
    xp = jnp.concatenate(
        [xf, jnp.broadcast_to(xf[0], (SEQ_PAD - xf.shape[0],))]).astype(jnp.int32)
    xb = (xp // SLAB) * SLAB
    xs = xp % SLAB
    wt = jnp.zeros((EMBED_DIM, LANES), jnp.float32).at[:, :W.shape[0]].set(W.T)
    bp = jnp.zeros((LANES,), jnp.float32).at[:b.shape[0]].set(b)
    res = _run(xb, xs, table.T, wt, bp)
    return res[:W.shape[0]].reshape(1, -1)


# speedup vs baseline: 19.0966x; 1.1711x over previous
import functools

import jax
import jax.numpy as jnp
from jax import lax
from jax.experimental import pallas as pl
from jax.experimental.pallas import tpu as pltpu
from jax.experimental.pallas import tpu_sc as plsc

SEQ_LEN = 200
IDX_PER_TILE = 16
NUM_TILES = 16
LAST_BASE = SEQ_LEN - IDX_PER_TILE
EMBED_DIM = 64
LANES = 16
SLAB = 128
WAVE_A = 8
WAVE_B = 6
NCHUNK = EMBED_DIM // LANES


def _sc_kernel(tt_hbm, x_hbm, wt_hbm, b_hbm, out_hbm,
               idx_v, slabs_a, slabs_b, lmax_v, tmp_v, wt_v, b_v, out_v,
               sem, *shared):
    cid = lax.axis_index("c")
    sid = lax.axis_index("s")

    @pl.when(cid == 0)
    def _gather_and_local_max():
        base = jnp.minimum(sid * IDX_PER_TILE, LAST_BASE)
        pltpu.sync_copy(x_hbm.at[pl.ds(base, IDX_PER_TILE)], idx_v)
        iv = idx_v[...]
        sv = jnp.bitwise_and(iv, SLAB - 1)
        bv = iv - sv

        def slab_dma(i, dst):
            return pltpu.async_copy(
                tt_hbm.at[:, pl.ds(pl.multiple_of(bv[i], SLAB), SLAB)],
                dst, sem)

        handles_a = [slab_dma(q, slabs_a.at[q]) for q in range(WAVE_A)]
        handles_b = [slab_dma(WAVE_A + q, slabs_b.at[q]) for q in range(WAVE_B)]
        for h in handles_a:
            h.wait()
        for h in handles_b:
            h.wait()

        lanes = lax.iota(jnp.int32, LANES)
        acc = [None] * NCHUNK

        def eat(i, slabs, q):
            i0 = jnp.broadcast_to(jnp.int32(q), (LANES,))
            i2 = jnp.broadcast_to(sv[i], (LANES,))
            for c in range(NCHUNK):
                g = plsc.load_gather(slabs, [i0, c * LANES + lanes, i2])
                acc[c] = g if acc[c] is None else jnp.maximum(acc[c], g)

        eat(0, slabs_a, 0)
        h14 = slab_dma(WAVE_A + WAVE_B, slabs_a.at[0])
        eat(1, slabs_a, 1)
        h15 = slab_dma(WAVE_A + WAVE_B + 1, slabs_a.at[1])
        for q in range(2, WAVE_A):
            eat(q, slabs_a, q)
        for q in range(WAVE_B):
            eat(WAVE_A + q, slabs_b, q)
        h14.wait()
        h15.wait()
        eat(WAVE_A + WAVE_B, slabs_a, 0)
        eat(WAVE_A + WAVE_B + 1, slabs_a, 1)

        for c in range(NCHUNK):
            lmax_v[pl.ds(c * LANES, LANES)] = acc[c]
        for k in range(NUM_TILES):
            @pl.when(sid == k)
            def _stage(k=k):
                pltpu.sync_copy(lmax_v, shared[k])

    plsc.subcore_barrier()

    @pl.when(jnp.logical_and(cid == 0, sid == 0))
    def _reduce_and_linear():
        pltpu.sync_copy(wt_hbm, wt_v)
        pltpu.sync_copy(b_hbm, b_v)
        gacc = [None] * NCHUNK
        for k in range(NUM_TILES):
            pltpu.sync_copy(shared[k], tmp_v)
            for c in range(NCHUNK):
                t = tmp_v[pl.ds(c * LANES, LANES)]
                gacc[c] = t if gacc[c] is None else jnp.maximum(gacc[c], t)
        acc = b_v[...]
        for c in range(NCHUNK):
            g = gacc[c]
            for l in range(LANES):
                acc = acc + g[l] * wt_v[c * LANES + l, :]
        out_v[...] = acc
        pltpu.sync_copy(out_v, out_hbm)


@jax.jit
def _run(xf, tt, wt, bp):
    mesh = plsc.VectorSubcoreMesh(core_axis_name="c", subcore_axis_name="s")
    k = functools.partial(
        pl.kernel,
        mesh=mesh,
        out_type=jax.ShapeDtypeStruct((LANES,), jnp.float32),
        compiler_params=pltpu.CompilerParams(needs_layout_passes=False),
        scratch_types=[
            pltpu.VMEM((IDX_PER_TILE,), jnp.int32),
            pltpu.VMEM((WAVE_A, EMBED_DIM, SLAB), jnp.float32),
            pltpu.VMEM((WAVE_B, EMBED_DIM, SLAB), jnp.float32),
            pltpu.VMEM((EMBED_DIM,), jnp.float32),
            pltpu.VMEM((EMBED_DIM,), jnp.float32),
            pltpu.VMEM((EMBED_DIM, LANES), jnp.float32),
            pltpu.VMEM((LANES,), jnp.float32),
            pltpu.VMEM((LANES,), jnp.float32),
            pltpu.SemaphoreType.DMA,
        ] + [pltpu.VMEM_SHARED((EMBED_DIM,), jnp.float32)
             for _ in range(NUM_TILES)],
    )(_sc_kernel)
    return k(tt, xf, wt, bp)


def kernel(x, table, W, b):
    xf = x.reshape(-1).astype(jnp.int32)
    wt = jnp.zeros((EMBED_DIM, LANES), jnp.float32).at[:, :W.shape[0]].set(W.T)
    bp = jnp.zeros((LANES,), jnp.float32).at[:b.shape[0]].set(b)
    res = _run(xf, table.T, wt, bp)
    return res[:W.shape[0]].reshape(1, -1)


# speedup vs baseline: 21.0970x; 1.1048x over previous
import functools

import jax
import jax.numpy as jnp
from jax import lax
from jax.experimental import pallas as pl
from jax.experimental.pallas import tpu as pltpu
from jax.experimental.pallas import tpu_sc as plsc

SEQ_LEN = 200
IDX_PER_TILE = 8
NUM_TILES = 16
NUM_CORES = 2
LAST_BASE = SEQ_LEN - IDX_PER_TILE
EMBED_DIM = 64
LANES = 16
SLAB = 128
NCHUNK = EMBED_DIM // LANES


def _sc_gather(tt_hbm, x_hbm, out0_hbm, out1_hbm,
               idx_v, slabs_v, lmax_v, tmp_v, out_v, sem, *shared):
    cid = lax.axis_index("c")
    sid = lax.axis_index("s")
    wid = cid * NUM_TILES + sid

    base = jnp.minimum(wid * IDX_PER_TILE, LAST_BASE)
    pltpu.sync_copy(x_hbm.at[pl.ds(base, IDX_PER_TILE)], idx_v)
    lanes = lax.iota(jnp.int32, LANES)
    iv = plsc.load_gather(idx_v, [jnp.minimum(lanes, IDX_PER_TILE - 1)])
    sv0 = jnp.bitwise_and(iv, SLAB - 1)
    bv = iv - sv0
    handles = [
        pltpu.async_copy(
            tt_hbm.at[:, pl.ds(pl.multiple_of(bv[q], SLAB), SLAB)],
            slabs_v.at[q], sem)
        for q in range(IDX_PER_TILE)
    ]
    for h in handles:
        h.wait()

    acc = [None] * NCHUNK
    for q in range(IDX_PER_TILE):
        i0 = jnp.broadcast_to(jnp.int32(q), (LANES,))
        i2 = jnp.broadcast_to(sv0[q], (LANES,))
        for c in range(NCHUNK):
            g = plsc.load_gather(slabs_v, [i0, c * LANES + lanes, i2])
            acc[c] = g if acc[c] is None else jnp.maximum(acc[c], g)
    for c in range(NCHUNK):
        lmax_v[pl.ds(c * LANES, LANES)] = acc[c]
    for k in range(NUM_TILES):
        @pl.when(sid == k)
        def _stage(k=k):
            pltpu.sync_copy(lmax_v, shared[k])

    plsc.subcore_barrier()

    @pl.when(sid == 0)
    def _reduce():
        gacc = [None] * NCHUNK
        for k in range(NUM_TILES):
            pltpu.sync_copy(shared[k], tmp_v)
            for c in range(NCHUNK):
                t = tmp_v[pl.ds(c * LANES, LANES)]
                gacc[c] = t if gacc[c] is None else jnp.maximum(gacc[c], t)
        for c in range(NCHUNK):
            out_v[0, pl.ds(c * LANES, LANES)] = gacc[c]

        @pl.when(cid == 0)
        def _w0():
            pltpu.sync_copy(out_v, out0_hbm)

        @pl.when(cid == 1)
        def _w1():
            pltpu.sync_copy(out_v, out1_hbm)


def _tc_combine(p0_ref, p1_ref, wt_ref, b_ref, o_ref):
    p = jnp.maximum(p0_ref[...], p1_ref[...])
    o_ref[...] = jnp.dot(
        p, wt_ref[...], preferred_element_type=jnp.float32) + b_ref[...]


@jax.jit
def _run(xf, tt, wtr, br):
    mesh = plsc.VectorSubcoreMesh(core_axis_name="c", subcore_axis_name="s")
    gather = functools.partial(
        pl.kernel,
        mesh=mesh,
        out_type=(jax.ShapeDtypeStruct((1, EMBED_DIM), jnp.float32),
                  jax.ShapeDtypeStruct((1, EMBED_DIM), jnp.float32)),
        compiler_params=pltpu.CompilerParams(needs_layout_passes=False),
        scratch_types=[
            pltpu.VMEM((IDX_PER_TILE,), jnp.int32),
            pltpu.VMEM((IDX_PER_TILE, EMBED_DIM, SLAB), jnp.float32),
            pltpu.VMEM((EMBED_DIM,), jnp.float32),
            pltpu.VMEM((EMBED_DIM,), jnp.float32),
            pltpu.VMEM((1, EMBED_DIM), jnp.float32),
            pltpu.SemaphoreType.DMA,
        ] + [pltpu.VMEM_SHARED((EMBED_DIM,), jnp.float32)
             for _ in range(NUM_TILES)],
    )(_sc_gather)
    p0, p1 = gather(tt, xf)
    out = pl.pallas_call(
        _tc_combine,
        out_shape=jax.ShapeDtypeStruct((1, 10), jnp.float32),
    )(p0, p1, wtr, br)
    return out


def kernel(x, table, W, b):
    xf = x.reshape(-1).astype(jnp.int32)
    return _run(xf, table.T, W.T, b.reshape(1, -1))


# speedup vs baseline: 22.3796x; 1.0608x over previous
import functools

import jax
import jax.numpy as jnp
from jax import lax
from jax.experimental import pallas as pl
from jax.experimental.pallas import tpu as pltpu
from jax.experimental.pallas import tpu_sc as plsc

SEQ_LEN = 200
IDX_PER_TILE = 8
NUM_TILES = 16
NUM_CORES = 2
ACTIVE_PER_CORE = 13
LAST_BASE = SEQ_LEN - IDX_PER_TILE
EMBED_DIM = 64
LANES = 16
SLAB = 128
NCHUNK = EMBED_DIM // LANES


def _sc_gather(tt_hbm, x_hbm, out0_hbm, out1_hbm,
               idx_v, slabs_v, lmax_v, tmp_v, out_v, sem, *shared):
    cid = lax.axis_index("c")
    sid = lax.axis_index("s")
    wid = sid * NUM_CORES + cid

    @pl.when(sid < ACTIVE_PER_CORE)
    def _gather_and_local_max():
        base = jnp.minimum(wid * IDX_PER_TILE, LAST_BASE)
        pltpu.sync_copy(x_hbm.at[pl.ds(base, IDX_PER_TILE)], idx_v)
        lanes = lax.iota(jnp.int32, LANES)
        iv = plsc.load_gather(idx_v, [jnp.minimum(lanes, IDX_PER_TILE - 1)])
        sv0 = jnp.bitwise_and(iv, SLAB - 1)
        bv = iv - sv0
        handles = [
            pltpu.async_copy(
                tt_hbm.at[:, pl.ds(pl.multiple_of(bv[q], SLAB), SLAB)],
                slabs_v.at[q], sem)
            for q in range(IDX_PER_TILE)
        ]
        for h in handles:
            h.wait()

        acc = [None] * NCHUNK
        for q in range(IDX_PER_TILE):
            i0 = jnp.broadcast_to(jnp.int32(q), (LANES,))
            i2 = jnp.broadcast_to(sv0[q], (LANES,))
            for c in range(NCHUNK):
                g = plsc.load_gather(slabs_v, [i0, c * LANES + lanes, i2])
                acc[c] = g if acc[c] is None else jnp.maximum(acc[c], g)
        for c in range(NCHUNK):
            lmax_v[pl.ds(c * LANES, LANES)] = acc[c]
        for k in range(ACTIVE_PER_CORE):
            @pl.when(sid == k)
            def _stage(k=k):
                pltpu.sync_copy(lmax_v, shared[k])

    plsc.subcore_barrier()

    @pl.when(sid == 0)
    def _reduce():
        gacc = [None] * NCHUNK
        for k in range(ACTIVE_PER_CORE):
            pltpu.sync_copy(shared[k], tmp_v)
            for c in range(NCHUNK):
                t = tmp_v[pl.ds(c * LANES, LANES)]
                gacc[c] = t if gacc[c] is None else jnp.maximum(gacc[c], t)
        for c in range(NCHUNK):
            out_v[0, pl.ds(c * LANES, LANES)] = gacc[c]

        @pl.when(cid == 0)
        def _w0():
            pltpu.sync_copy(out_v, out0_hbm)

        @pl.when(cid == 1)
        def _w1():
            pltpu.sync_copy(out_v, out1_hbm)


def _tc_combine(p0_ref, p1_ref, wt_ref, b_ref, o_ref):
    p = jnp.maximum(p0_ref[...], p1_ref[...])
    o_ref[...] = jnp.dot(
        p, wt_ref[...], preferred_element_type=jnp.float32) + b_ref[...]


@jax.jit
def _run(xf, tt, wtr, br):
    mesh = plsc.VectorSubcoreMesh(core_axis_name="c", subcore_axis_name="s")
    gather = functools.partial(
        pl.kernel,
        mesh=mesh,
        out_type=(jax.ShapeDtypeStruct((1, EMBED_DIM), jnp.float32),
                  jax.ShapeDtypeStruct((1, EMBED_DIM), jnp.float32)),
        compiler_params=pltpu.CompilerParams(needs_layout_passes=False),
        scratch_types=[
            pltpu.VMEM((IDX_PER_TILE,), jnp.int32),
            pltpu.VMEM((IDX_PER_TILE, EMBED_DIM, SLAB), jnp.float32),
            pltpu.VMEM((EMBED_DIM,), jnp.float32),
            pltpu.VMEM((EMBED_DIM,), jnp.float32),
            pltpu.VMEM((1, EMBED_DIM), jnp.float32),
            pltpu.SemaphoreType.DMA,
        ] + [pltpu.VMEM_SHARED((EMBED_DIM,), jnp.float32)
             for _ in range(NUM_TILES)],
    )(_sc_gather)
    p0, p1 = gather(tt, xf)
    out = pl.pallas_call(
        _tc_combine,
        out_shape=jax.ShapeDtypeStruct((1, 10), jnp.float32),
    )(p0, p1, wtr, br)
    return out


def kernel(x, table, W, b):
    xf = x.reshape(-1).astype(jnp.int32)
    return _run(xf, table.T, W.T, b.reshape(1, -1))
